# baseline (speedup 1.0000x reference)
import functools

import jax
import jax.numpy as jnp
from jax import lax
from jax.experimental import pallas as pl
from jax.experimental.pallas import tpu as pltpu
from jax.experimental.pallas import tpu_sc as plsc

NUM_EMB = 100000
D = 128
BATCH = 4096
HIST = 50
TOTAL = BATCH * HIST

NC = 2
NS = 16
NW = NC * NS
PER_W = TOTAL // NW
CHUNK = 256
NCH = PER_W // CHUNK
NBUF = 3


@functools.partial(
    pl.kernel,
    mesh=plsc.VectorSubcoreMesh(core_axis_name="c", subcore_axis_name="s"),
    out_type=jax.ShapeDtypeStruct((TOTAL, D), jnp.float32),
    scratch_types=[
        pltpu.VMEM((PER_W,), jnp.int32),
        pltpu.VMEM((NBUF, CHUNK, D), jnp.float32),
        pltpu.SemaphoreType.DMA,
        pltpu.SemaphoreType.DMA,
    ],
)
def _emb_gather(idx_hbm, table_hbm, out_hbm, idx_v, buf, gsem, ssem):
    wid = lax.axis_index("s") * NC + lax.axis_index("c")
    base = wid * PER_W
    pltpu.sync_copy(idx_hbm.at[pl.ds(base, PER_W)], idx_v)

    for b in range(NBUF - 1):
        pltpu.async_copy(table_hbm.at[idx_v.at[pl.ds(b * CHUNK, CHUNK)]], buf.at[b], gsem)

    def step(g, carry):
        slot = lax.rem(g, NBUF)
        pltpu.make_async_copy(
            table_hbm.at[idx_v.at[pl.ds(g * CHUNK, CHUNK)]], buf.at[slot], gsem
        ).wait()

        @pl.when(g + NBUF - 1 < NCH)
        def _():
            nslot = lax.rem(g + NBUF - 1, NBUF)

            @pl.when(g >= 1)
            def _():
                pltpu.make_async_copy(
                    buf.at[nslot],
                    out_hbm.at[pl.ds(base + (g - 1) * CHUNK, CHUNK)],
                    ssem,
                ).wait()

            pltpu.async_copy(
                table_hbm.at[idx_v.at[pl.ds((g + NBUF - 1) * CHUNK, CHUNK)]], buf.at[nslot], gsem
            )

        pltpu.async_copy(
            buf.at[slot], out_hbm.at[pl.ds(base + g * CHUNK, CHUNK)], ssem
        )
        return carry

    lax.fori_loop(0, NCH, step, 0)

    for g in range(NCH - NBUF, NCH):
        pltpu.make_async_copy(
            buf.at[g % NBUF], out_hbm.at[pl.ds(base + g * CHUNK, CHUNK)], ssem
        ).wait()


def kernel(token_ids, lookup):
    idx_flat = token_ids.T.reshape(TOTAL).astype(jnp.int32)
    out = _emb_gather(idx_flat, lookup)
    return out.reshape(HIST, BATCH, D).transpose(1, 0, 2)


# speedup vs baseline: 1.0000x; 1.0000x over previous
import functools

import jax
import jax.numpy as jnp
from jax import lax
from jax.experimental import pallas as pl
from jax.experimental.pallas import tpu as pltpu
from jax.experimental.pallas import tpu_sc as plsc

NUM_EMB = 100000
D = 128
BATCH = 4096
HIST = 50
TOTAL = BATCH * HIST

NC = 2
NS = 16
NW = NC * NS
PER_W = TOTAL // NW
CHUNK = 128
NCH = PER_W // CHUNK
NBUF = 6


@functools.partial(
    pl.kernel,
    mesh=plsc.VectorSubcoreMesh(core_axis_name="c", subcore_axis_name="s"),
    out_type=jax.ShapeDtypeStruct((TOTAL, D), jnp.float32),
    scratch_types=[
        pltpu.VMEM((PER_W,), jnp.int32),
        pltpu.VMEM((NBUF, CHUNK, D), jnp.float32),
        pltpu.SemaphoreType.DMA,
        pltpu.SemaphoreType.DMA,
        pltpu.SemaphoreType.DMA,
    ],
)
def _emb_gather(idx_hbm, table_hbm, out_hbm, idx_v, buf, gsem_a, gsem_b, ssem):
    wid = lax.axis_index("s") * NC + lax.axis_index("c")
    base = wid * PER_W
    pltpu.sync_copy(idx_hbm.at[pl.ds(base, PER_W)], idx_v)

    gsems = (gsem_a, gsem_b)
    for b in range(NBUF - 1):
        pltpu.async_copy(
            table_hbm.at[idx_v.at[pl.ds(b * CHUNK, CHUNK)]], buf.at[b], gsems[b % 2]
        )

    def step(g, carry):
        slot = lax.rem(g, NBUF)

        @pl.when(lax.rem(g, 2) == 0)
        def _():
            pltpu.make_async_copy(
                table_hbm.at[idx_v.at[pl.ds(g * CHUNK, CHUNK)]], buf.at[slot], gsem_a
            ).wait()

        @pl.when(lax.rem(g, 2) == 1)
        def _():
            pltpu.make_async_copy(
                table_hbm.at[idx_v.at[pl.ds(g * CHUNK, CHUNK)]], buf.at[slot], gsem_b
            ).wait()

        @pl.when(g + NBUF - 1 < NCH)
        def _():
            nslot = lax.rem(g + NBUF - 1, NBUF)

            @pl.when(g >= 1)
            def _():
                pltpu.make_async_copy(
                    buf.at[nslot],
                    out_hbm.at[pl.ds(base + (g - 1) * CHUNK, CHUNK)],
                    ssem,
                ).wait()

            gnext = g + NBUF - 1

            @pl.when(lax.rem(gnext, 2) == 0)
            def _():
                pltpu.async_copy(
                    table_hbm.at[idx_v.at[pl.ds(gnext * CHUNK, CHUNK)]], buf.at[nslot], gsem_a
                )

            @pl.when(lax.rem(gnext, 2) == 1)
            def _():
                pltpu.async_copy(
                    table_hbm.at[idx_v.at[pl.ds(gnext * CHUNK, CHUNK)]], buf.at[nslot], gsem_b
                )

        pltpu.async_copy(
            buf.at[slot], out_hbm.at[pl.ds(base + g * CHUNK, CHUNK)], ssem
        )
        return carry

    lax.fori_loop(0, NCH, step, 0)

    for g in range(NCH - NBUF, NCH):
        pltpu.make_async_copy(
            buf.at[g % NBUF], out_hbm.at[pl.ds(base + g * CHUNK, CHUNK)], ssem
        ).wait()


def kernel(token_ids, lookup):
    idx_flat = token_ids.T.reshape(TOTAL).astype(jnp.int32)
    out = _emb_gather(idx_flat, lookup)
    return out.reshape(HIST, BATCH, D).transpose(1, 0, 2)


# speedup vs baseline: 1.0187x; 1.0187x over previous
import functools

import jax
import jax.numpy as jnp
from jax import lax
from jax.experimental import pallas as pl
from jax.experimental.pallas import tpu as pltpu
from jax.experimental.pallas import tpu_sc as plsc

NUM_EMB = 100000
D = 128
BATCH = 4096
HIST = 50

NC = 2
NS = 16
NW = NC * NS
COLS_W = BATCH // NW
NBUF = 4


@functools.partial(
    pl.kernel,
    mesh=plsc.VectorSubcoreMesh(core_axis_name="c", subcore_axis_name="s"),
    out_type=jax.ShapeDtypeStruct((HIST, BATCH, D), jnp.float32),
    scratch_types=[
        pltpu.VMEM((HIST, COLS_W), jnp.int32),
        pltpu.VMEM((NBUF, COLS_W, D), jnp.float32),
        pltpu.SemaphoreType.DMA,
        pltpu.SemaphoreType.DMA,
    ],
)
def _emb_gather(idx_hbm, table_hbm, out_hbm, idx_v, buf, gsem, ssem):
    wid = lax.axis_index("s") * NC + lax.axis_index("c")
    base = wid * COLS_W
    pltpu.sync_copy(idx_hbm.at[:, pl.ds(base, COLS_W)], idx_v)

    for b in range(NBUF - 1):
        pltpu.async_copy(table_hbm.at[idx_v.at[b]], buf.at[b], gsem)

    def step(g, carry):
        slot = lax.rem(g, NBUF)
        pltpu.make_async_copy(
            table_hbm.at[idx_v.at[g]], buf.at[slot], gsem
        ).wait()

        @pl.when(g + NBUF - 1 < HIST)
        def _():
            nslot = lax.rem(g + NBUF - 1, NBUF)

            @pl.when(g >= 1)
            def _():
                pltpu.make_async_copy(
                    buf.at[nslot],
                    out_hbm.at[g - 1, pl.ds(base, COLS_W)],
                    ssem,
                ).wait()

            pltpu.async_copy(
                table_hbm.at[idx_v.at[g + NBUF - 1]], buf.at[nslot], gsem
            )

        pltpu.async_copy(
            buf.at[slot], out_hbm.at[g, pl.ds(base, COLS_W)], ssem
        )
        return carry

    lax.fori_loop(0, HIST, step, 0)

    for g in range(HIST - NBUF, HIST):
        pltpu.make_async_copy(
            buf.at[g % NBUF], out_hbm.at[g, pl.ds(base, COLS_W)], ssem
        ).wait()


def kernel(token_ids, lookup):
    out = _emb_gather(token_ids.T.astype(jnp.int32), lookup)
    return out.transpose(1, 0, 2)


# speedup vs baseline: 1.0192x; 1.0005x over previous
import functools

import jax
import jax.numpy as jnp
from jax import lax
from jax.experimental import pallas as pl
from jax.experimental.pallas import tpu as pltpu
from jax.experimental.pallas import tpu_sc as plsc

NUM_EMB = 100000
D = 128
BATCH = 4096
HIST = 50

NC = 2
NS = 16
NW = NC * NS
COLS_W = BATCH // NW
NBUF = 4


@functools.partial(
    pl.kernel,
    mesh=plsc.VectorSubcoreMesh(core_axis_name="c", subcore_axis_name="s"),
    compiler_params=pltpu.CompilerParams(skip_device_barrier=True),
    out_type=jax.ShapeDtypeStruct((HIST, BATCH, D), jnp.float32),
    scratch_types=[
        pltpu.VMEM((HIST, COLS_W), jnp.int32),
        pltpu.VMEM((NBUF, COLS_W, D), jnp.float32),
        pltpu.SemaphoreType.DMA,
        pltpu.SemaphoreType.DMA,
    ],
)
def _emb_gather(idx_hbm, table_hbm, out_hbm, idx_v, buf, gsem, ssem):
    wid = lax.axis_index("s") * NC + lax.axis_index("c")
    base = wid * COLS_W
    pltpu.sync_copy(idx_hbm.at[:, pl.ds(base, COLS_W)], idx_v)

    for b in range(NBUF - 1):
        pltpu.async_copy(table_hbm.at[idx_v.at[b]], buf.at[b], gsem)

    def step(g, carry):
        slot = lax.rem(g, NBUF)
        pltpu.make_async_copy(
            table_hbm.at[idx_v.at[g]], buf.at[slot], gsem
        ).wait()

        @pl.when(g + NBUF - 1 < HIST)
        def _():
            nslot = lax.rem(g + NBUF - 1, NBUF)

            @pl.when(g >= 1)
            def _():
                pltpu.make_async_copy(
                    buf.at[nslot],
                    out_hbm.at[g - 1, pl.ds(base, COLS_W)],
                    ssem,
                ).wait()

            pltpu.async_copy(
                table_hbm.at[idx_v.at[g + NBUF - 1]], buf.at[nslot], gsem
            )

        pltpu.async_copy(
            buf.at[slot], out_hbm.at[g, pl.ds(base, COLS_W)], ssem
        )
        return carry

    lax.fori_loop(0, HIST, step, 0)

    for g in range(HIST - NBUF, HIST):
        pltpu.make_async_copy(
            buf.at[g % NBUF], out_hbm.at[g, pl.ds(base, COLS_W)], ssem
        ).wait()


def kernel(token_ids, lookup):
    out = _emb_gather(token_ids.T.astype(jnp.int32), lookup)
    return out.transpose(1, 0, 2)
